# Initial kernel scaffold; baseline (speedup 1.0000x reference)
import jax
import jax.numpy as jnp
from jax.experimental import pallas as pl


def kernel(train, test_items):
    raise NotImplementedError("write your pallas kernel here")


# speedup vs baseline: 3.3029x; 3.3029x over previous
import functools

import jax
import jax.numpy as jnp
from jax import lax
from jax.experimental import pallas as pl
from jax.experimental.pallas import tpu as pltpu
from jax.experimental.pallas import tpu_sc as plsc

_COL_BLOCK = 2048
_LANES = 16


def _colsum_body(train_ref, score_ref):
    score_ref[...] = jnp.sum(train_ref[...], axis=0, keepdims=True)


def _colsum(train):
    n_rows, n_cols = train.shape
    return pl.pallas_call(
        _colsum_body,
        grid=(pl.cdiv(n_cols, _COL_BLOCK),),
        in_specs=[pl.BlockSpec((n_rows, _COL_BLOCK), lambda j: (0, j))],
        out_specs=pl.BlockSpec((1, _COL_BLOCK), lambda j: (0, j)),
        out_shape=jax.ShapeDtypeStruct((1, n_cols), jnp.float32),
    )(train)


@functools.cache
def _make_gather(n_items, n_idx):
    info = plsc.get_sparse_core_info()
    n_workers = info.num_cores * info.num_subcores
    per_w = n_idx // n_workers
    assert per_w * n_workers == n_idx and per_w % _LANES == 0
    mesh = plsc.VectorSubcoreMesh(core_axis_name="c", subcore_axis_name="s")

    @functools.partial(
        pl.kernel,
        mesh=mesh,
        out_type=jax.ShapeDtypeStruct((n_idx,), jnp.float32),
        scratch_types=[
            pltpu.VMEM((n_items,), jnp.float32),
            pltpu.VMEM((per_w,), jnp.int32),
            pltpu.VMEM((per_w,), jnp.float32),
        ],
        compiler_params=pltpu.CompilerParams(needs_layout_passes=False),
    )
    def gather_kernel(score_hbm, idx_hbm, out_hbm, table_v, idx_v, out_v):
        wid = lax.axis_index("s") * info.num_cores + lax.axis_index("c")
        base = wid * per_w
        pltpu.sync_copy(score_hbm, table_v)
        pltpu.sync_copy(idx_hbm.at[pl.ds(base, per_w)], idx_v)

        def body(i, carry):
            iv = idx_v[pl.ds(i * _LANES, _LANES)]
            out_v[pl.ds(i * _LANES, _LANES)] = plsc.load_gather(table_v, [iv])
            return carry

        lax.fori_loop(0, per_w // _LANES, body, 0)
        pltpu.sync_copy(out_v, out_hbm.at[pl.ds(base, per_w)])

    return gather_kernel


def kernel(train, test_items):
    score = _colsum(train)[0]
    idx = test_items.reshape(-1).astype(jnp.int32)
    out = _make_gather(score.shape[0], idx.shape[0])(score, idx)
    return out.reshape(test_items.shape)
